# baseline (speedup 1.0000x reference)
import functools

import jax
import jax.numpy as jnp
from jax import lax
from jax.experimental import pallas as pl
from jax.experimental.pallas import tpu as pltpu
from jax.experimental.pallas import tpu_sc as plsc

_B, _N, _K = 2, 40960, 16
_NK = _N * _K
_Q = _B * _NK
_EPS = 1e-5

_NC, _NS = 2, 16
_NW = _NC * _NS
_C = 2048
_CR = _C // 128

_CB = 4096
_NB = _NK // _CB
_PB = _CB // _K


def _sc_gather(xt, yt, zt, idx_hbm, xo, yo, zo, idx_v, xg, yg, zg, sem):
    wid = lax.axis_index("s") * _NC + lax.axis_index("c")
    per_w = _Q // _NW
    num_chunks = per_w // _C

    def chunk_body(t, carry):
        base = pl.multiple_of(wid * per_w + t * _C, _C)
        pltpu.sync_copy(idx_hbm.at[pl.ds(base, _C)], idx_v)
        c1 = pltpu.async_copy(xt.at[idx_v], xg, sem)
        c2 = pltpu.async_copy(yt.at[idx_v], yg, sem)
        c3 = pltpu.async_copy(zt.at[idx_v], zg, sem)
        c1.wait()
        c2.wait()
        c3.wait()
        pltpu.sync_copy(xg, xo.at[pl.ds(base, _C)])
        pltpu.sync_copy(yg, yo.at[pl.ds(base, _C)])
        pltpu.sync_copy(zg, zo.at[pl.ds(base, _C)])
        return carry

    lax.fori_loop(0, num_chunks, chunk_body, 0)


def _feat_block(xr, yr, zr, ptsr, er):
    nbr = jnp.concatenate([xr[0], yr[0], zr[0]], axis=0)
    pts4 = ptsr[0]
    pts = lax.dot_general(pts4, er[...], (((1,), (0,)), ((), ())),
                          preferred_element_type=jnp.float32,
                          precision=lax.Precision.HIGHEST)[0:3]
    rel = pts - nbr
    dist = jnp.sqrt(jnp.sum(rel * rel, axis=0, keepdims=True))
    ones = jnp.ones((1, _CB), jnp.float32)
    return jnp.concatenate([dist, pts, nbr, ones], axis=0)


def _moments_body(xr, yr, zr, ptsr, er, out_ref):
    f8 = _feat_block(xr, yr, zr, ptsr, er)
    part = lax.dot_general(f8, f8, (((1,), (1,)), ((), ())),
                           preferred_element_type=jnp.float32,
                           precision=lax.Precision.HIGHEST)

    @pl.when((pl.program_id(0) == 0) & (pl.program_id(1) == 0))
    def _init():
        out_ref[...] = jnp.zeros_like(out_ref)

    out_ref[...] += part


def _apply_body(xr, yr, zr, ptsr, er, wt_ref, out_ref):
    f8 = _feat_block(xr, yr, zr, ptsr, er)
    o = lax.dot_general(wt_ref[...], f8, (((1,), (0,)), ((), ())),
                        preferred_element_type=jnp.float32,
                        precision=lax.Precision.HIGHEST)
    out_ref[0] = jnp.maximum(o, 0.0)


def kernel(xyz, feature, neigh_idx, W, b, gamma, beta):
    del feature
    xt = xyz[:, :, 0].reshape(_B * _N)
    yt = xyz[:, :, 1].reshape(_B * _N)
    zt = xyz[:, :, 2].reshape(_B * _N)
    idx_glob = (neigh_idx.reshape(_B, _NK)
                + (jnp.arange(_B, dtype=jnp.int32) * _N)[:, None])
    idx_flat = idx_glob.reshape(_Q)
    xyz_t = jnp.concatenate(
        [jnp.transpose(xyz, (0, 2, 1)),
         jnp.zeros((_B, 1, _N), jnp.float32)], axis=1)

    mesh = plsc.VectorSubcoreMesh(core_axis_name="c", subcore_axis_name="s")
    vshape = jax.ShapeDtypeStruct((_Q,), jnp.float32)
    xo, yo, zo = pl.kernel(
        _sc_gather,
        mesh=mesh,
        out_type=(vshape, vshape, vshape),
        scratch_types=[
            pltpu.VMEM((_C,), jnp.int32),
            pltpu.VMEM((_C,), jnp.float32),
            pltpu.VMEM((_C,), jnp.float32),
            pltpu.VMEM((_C,), jnp.float32),
            pltpu.SemaphoreType.DMA,
        ],
    )(xt, yt, zt, idx_flat)
    xo = xo.reshape(_B, 1, _NK)
    yo = yo.reshape(_B, 1, _NK)
    zo = zo.reshape(_B, 1, _NK)

    w_dist = W[0:1]
    wx = W[1:4] + W[4:7]
    wn = W[7:10] - W[1:4]
    W_aug = jnp.concatenate([w_dist, wx, wn, b[None, :]], axis=0)

    nbr_specs = [pl.BlockSpec((1, 1, _CB), lambda bb, i: (bb, 0, i))
                 for _ in range(3)]
    pts_spec = pl.BlockSpec((1, 4, _PB), lambda bb, i: (bb, 0, i))
    expand_mat = (jnp.arange(_PB, dtype=jnp.int32)[:, None]
                  == (jnp.arange(_CB, dtype=jnp.int32)[None, :] // _K)
                  ).astype(jnp.float32)
    e_spec = pl.BlockSpec((_PB, _CB), lambda bb, i: (0, 0))

    m2 = pl.pallas_call(
        _moments_body,
        grid=(_B, _NB),
        in_specs=nbr_specs + [pts_spec, e_spec],
        out_specs=pl.BlockSpec((8, 8), lambda bb, i: (0, 0)),
        out_shape=jax.ShapeDtypeStruct((8, 8), jnp.float32),
    )(xo, yo, zo, xyz_t, expand_mat)

    p_cnt = float(_Q)
    s1 = m2[:, 7]
    mean = (s1 / p_cnt) @ W_aug
    ex2 = jnp.einsum("fc,fg,gc->c", W_aug, m2, W_aug) / p_cnt
    var = jnp.maximum(ex2 - mean * mean, 0.0)
    scale = gamma / jnp.sqrt(var + _EPS)
    w_fin = jnp.concatenate(
        [W_aug[0:7] * scale[None, :],
         ((W_aug[7] - mean) * scale + beta)[None, :]], axis=0)

    out = pl.pallas_call(
        _apply_body,
        grid=(_B, _NB),
        in_specs=nbr_specs + [
            pts_spec,
            e_spec,
            pl.BlockSpec((16, 8), lambda bb, i: (0, 0)),
        ],
        out_specs=pl.BlockSpec((1, 16, _CB), lambda bb, i: (bb, 0, i)),
        out_shape=jax.ShapeDtypeStruct((_B, 16, _NK), jnp.float32),
    )(xo, yo, zo, xyz_t, expand_mat, w_fin.T)

    return out.reshape(_B, 16, _N, _K)


# speedup vs baseline: 1.5600x; 1.5600x over previous
import functools

import jax
import jax.numpy as jnp
from jax import lax
from jax.experimental import pallas as pl
from jax.experimental.pallas import tpu as pltpu
from jax.experimental.pallas import tpu_sc as plsc

_B, _N, _K = 2, 40960, 16
_NK = _N * _K
_Q = _B * _NK
_EPS = 1e-5

_NC, _NS = 2, 16
_NW = _NC * _NS
_C = 2048
_CR = _C // 128

_CB = 4096
_NB = _NK // _CB
_PB = _CB // _K


def _sc_gather(xt, yt, zt, idx_hbm, xo, yo, zo, idx_v, xg, yg, zg, sem):
    wid = lax.axis_index("s") * _NC + lax.axis_index("c")
    per_w = _Q // _NW
    num_chunks = per_w // _C

    def chunk_body(t, carry):
        base = pl.multiple_of(wid * per_w + t * _C, _C)
        pltpu.sync_copy(idx_hbm.at[pl.ds(base, _C)], idx_v)
        c1 = pltpu.async_copy(xt.at[idx_v], xg, sem)
        c2 = pltpu.async_copy(yt.at[idx_v], yg, sem)
        c3 = pltpu.async_copy(zt.at[idx_v], zg, sem)
        c1.wait()
        c2.wait()
        c3.wait()
        pltpu.sync_copy(xg, xo.at[pl.ds(base, _C)])
        pltpu.sync_copy(yg, yo.at[pl.ds(base, _C)])
        pltpu.sync_copy(zg, zo.at[pl.ds(base, _C)])
        return carry

    lax.fori_loop(0, num_chunks, chunk_body, 0)


def _feat_block(xr, yr, zr, ptsr):
    nbr = jnp.concatenate([xr[0], yr[0], zr[0]], axis=0)
    pts4 = ptsr[0][0:3]
    pts = jnp.broadcast_to(pts4[:, :, None], (3, _PB, _K)).reshape(3, _CB)
    rel = pts - nbr
    dist = jnp.sqrt(jnp.sum(rel * rel, axis=0, keepdims=True))
    ones = jnp.ones((1, _CB), jnp.float32)
    return jnp.concatenate([dist, pts, nbr, ones], axis=0)


def _moments_body(xr, yr, zr, ptsr, out_ref):
    f8 = _feat_block(xr, yr, zr, ptsr)
    part = lax.dot_general(f8, f8, (((1,), (1,)), ((), ())),
                           preferred_element_type=jnp.float32,
                           precision=lax.Precision.HIGHEST)

    @pl.when((pl.program_id(0) == 0) & (pl.program_id(1) == 0))
    def _init():
        out_ref[...] = jnp.zeros_like(out_ref)

    out_ref[...] += part


def _apply_body(xr, yr, zr, ptsr, wt_ref, out_ref):
    f8 = _feat_block(xr, yr, zr, ptsr)
    o = lax.dot_general(wt_ref[...], f8, (((1,), (0,)), ((), ())),
                        preferred_element_type=jnp.float32,
                        precision=lax.Precision.HIGHEST)
    out_ref[0] = jnp.maximum(o, 0.0)


def kernel(xyz, feature, neigh_idx, W, b, gamma, beta):
    del feature
    xt = xyz[:, :, 0].reshape(_B * _N)
    yt = xyz[:, :, 1].reshape(_B * _N)
    zt = xyz[:, :, 2].reshape(_B * _N)
    idx_glob = (neigh_idx.reshape(_B, _NK)
                + (jnp.arange(_B, dtype=jnp.int32) * _N)[:, None])
    idx_flat = idx_glob.reshape(_Q)
    xyz_t = jnp.concatenate(
        [jnp.transpose(xyz, (0, 2, 1)),
         jnp.zeros((_B, 1, _N), jnp.float32)], axis=1)

    mesh = plsc.VectorSubcoreMesh(core_axis_name="c", subcore_axis_name="s")
    vshape = jax.ShapeDtypeStruct((_Q,), jnp.float32)
    xo, yo, zo = pl.kernel(
        _sc_gather,
        mesh=mesh,
        out_type=(vshape, vshape, vshape),
        scratch_types=[
            pltpu.VMEM((_C,), jnp.int32),
            pltpu.VMEM((_C,), jnp.float32),
            pltpu.VMEM((_C,), jnp.float32),
            pltpu.VMEM((_C,), jnp.float32),
            pltpu.SemaphoreType.DMA,
        ],
    )(xt, yt, zt, idx_flat)
    xo = xo.reshape(_B, 1, _NK)
    yo = yo.reshape(_B, 1, _NK)
    zo = zo.reshape(_B, 1, _NK)

    w_dist = W[0:1]
    wx = W[1:4] + W[4:7]
    wn = W[7:10] - W[1:4]
    W_aug = jnp.concatenate([w_dist, wx, wn, b[None, :]], axis=0)

    nbr_specs = [pl.BlockSpec((1, 1, _CB), lambda bb, i: (bb, 0, i))
                 for _ in range(3)]
    pts_spec = pl.BlockSpec((1, 4, _PB), lambda bb, i: (bb, 0, i))

    m2 = pl.pallas_call(
        _moments_body,
        grid=(_B, _NB),
        in_specs=nbr_specs + [pts_spec],
        out_specs=pl.BlockSpec((8, 8), lambda bb, i: (0, 0)),
        out_shape=jax.ShapeDtypeStruct((8, 8), jnp.float32),
    )(xo, yo, zo, xyz_t)

    p_cnt = float(_Q)
    s1 = m2[:, 7]
    mean = (s1 / p_cnt) @ W_aug
    ex2 = jnp.einsum("fc,fg,gc->c", W_aug, m2, W_aug) / p_cnt
    var = jnp.maximum(ex2 - mean * mean, 0.0)
    scale = gamma / jnp.sqrt(var + _EPS)
    w_fin = jnp.concatenate(
        [W_aug[0:7] * scale[None, :],
         ((W_aug[7] - mean) * scale + beta)[None, :]], axis=0)

    out = pl.pallas_call(
        _apply_body,
        grid=(_B, _NB),
        in_specs=nbr_specs + [
            pts_spec,
            pl.BlockSpec((16, 8), lambda bb, i: (0, 0)),
        ],
        out_specs=pl.BlockSpec((1, 16, _CB), lambda bb, i: (bb, 0, i)),
        out_shape=jax.ShapeDtypeStruct((_B, 16, _NK), jnp.float32),
    )(xo, yo, zo, xyz_t, w_fin.T)

    return out.reshape(_B, 16, _N, _K)
